# baseline (speedup 1.0000x reference)
import functools

import jax
import jax.numpy as jnp
from jax import lax
from jax.experimental import pallas as pl
from jax.experimental.pallas import tpu as pltpu
from jax.experimental.pallas import tpu_sc as plsc

NC = 2
NS = 16
L = 16
NW = NC * NS

D = 128
CHUNK = 128
NBC = 8
MB = 1280
N_PAD = 10240
RPS = N_PAD // NS


def _mesh():
    return plsc.VectorSubcoreMesh(
        core_axis_name="c", subcore_axis_name="s", num_cores=NC, num_subcores=NS
    )


@functools.lru_cache(maxsize=None)
def _sc_degree(e_pad):
    ept = e_pad // NW
    nch = ept // CHUNK

    @functools.partial(
        pl.kernel,
        out_type=jax.ShapeDtypeStruct((NC * N_PAD,), jnp.float32),
        mesh=_mesh(),
        scratch_types=[
            pltpu.VMEM((CHUNK,), jnp.int32),
            pltpu.VMEM((CHUNK,), jnp.float32),
            pltpu.VMEM((RPS,), jnp.float32),
            pltpu.VMEM_SHARED((N_PAD,), jnp.float32),
        ],
    )
    def deg_kernel(row_hbm, out_hbm, idx_v, ones_v, zero_v, s1):
        c = lax.axis_index("c")
        s = lax.axis_index("s")
        wid = s * NC + c
        ones = jnp.ones((L,), jnp.float32)
        zeros = jnp.zeros((L,), jnp.float32)
        for j in range(CHUNK // L):
            ones_v[pl.ds(j * L, L)] = ones

        def zfill(i, _):
            zero_v[pl.ds(i * L, L)] = zeros
            return 0

        lax.fori_loop(0, RPS // L, zfill, 0)
        pltpu.sync_copy(zero_v, s1.at[pl.ds(s * RPS, RPS)])
        plsc.subcore_barrier()

        def chunk(j, _):
            off = pl.multiple_of(wid * ept + j * CHUNK, CHUNK)
            pltpu.sync_copy(row_hbm.at[pl.ds(off, CHUNK)], idx_v)
            pltpu.sync_copy(ones_v, s1.at[idx_v], add=True)
            return 0

        lax.fori_loop(0, nch, chunk, 0)
        plsc.subcore_barrier()
        pltpu.sync_copy(
            s1.at[pl.ds(s * RPS, RPS)],
            out_hbm.at[pl.ds(c * N_PAD + s * RPS, RPS)],
        )

    return deg_kernel


@functools.lru_cache(maxsize=None)
def _sc_segsum(e_pad):
    totch = e_pad // CHUNK
    nch = totch // NW

    @functools.partial(
        pl.kernel,
        out_type=jax.ShapeDtypeStruct((NC, N_PAD, D), jnp.float32),
        mesh=_mesh(),
        scratch_types=[
            pltpu.VMEM((CHUNK,), jnp.int32),
            pltpu.VMEM((CHUNK,), jnp.int32),
            pltpu.VMEM((CHUNK, D), jnp.float32),
            pltpu.VMEM_SHARED((N_PAD, D), jnp.float32),
            pltpu.SemaphoreType.DMA,
        ],
    )
    def seg_kernel(
        g_hbm, row_hbm, col_hbm, zeros_hbm, out_hbm,
        ridx_v, cidx_v, rows_v, sacc, sem,
    ):
        c = lax.axis_index("c")
        s = lax.axis_index("s")
        wid = s * NC + c
        for j in range(RPS // CHUNK):
            pltpu.sync_copy(
                zeros_hbm, sacc.at[pl.ds(s * RPS + j * CHUNK, CHUNK)]
            )
        plsc.subcore_barrier()

        def chunk(j, _):
            off = pl.multiple_of((wid * nch + j) * CHUNK, CHUNK)
            pltpu.sync_copy(row_hbm.at[pl.ds(off, CHUNK)], ridx_v)
            pltpu.sync_copy(col_hbm.at[pl.ds(off, CHUNK)], cidx_v)
            pltpu.async_copy(g_hbm.at[cidx_v], rows_v, sem).wait()
            pltpu.sync_copy(rows_v, sacc.at[ridx_v], add=True)
            return 0

        lax.fori_loop(0, nch, chunk, 0)
        plsc.subcore_barrier()
        pltpu.sync_copy(
            sacc.at[pl.ds(s * RPS, RPS)], out_hbm.at[c, pl.ds(s * RPS, RPS)]
        )

    return seg_kernel


def _tc_g_body(x_ref, w_ref, db_ref, g_ref):
    db = db_ref[...]
    deg = 1.0 + db[:, 0] + db[:, 1]
    d = lax.rsqrt(deg)
    h = jnp.dot(x_ref[...], w_ref[...], preferred_element_type=jnp.float32)
    g_ref[...] = h * d[:, None]


_tc_g = pl.pallas_call(
    _tc_g_body,
    grid=(N_PAD // MB,),
    in_specs=[
        pl.BlockSpec((MB, D), lambda i: (i, 0)),
        pl.BlockSpec((D, D), lambda i: (0, 0)),
        pl.BlockSpec((MB, NC), lambda i: (i, 0)),
    ],
    out_specs=pl.BlockSpec((MB, D), lambda i: (i, 0)),
    out_shape=jax.ShapeDtypeStruct((N_PAD, D), jnp.float32),
)


def _tc_out_body(s_ref, g_ref, db_ref, b_ref, o_ref):
    db = db_ref[...]
    deg = 1.0 + db[:, 0] + db[:, 1]
    d = lax.rsqrt(deg)
    sv = s_ref[...]
    tot = sv[0] + sv[1] + g_ref[...]
    o_ref[...] = tot * d[:, None] + b_ref[...]


def _tc_out(n_nodes, ob):
    return pl.pallas_call(
        _tc_out_body,
        grid=(n_nodes // ob,),
        in_specs=[
            pl.BlockSpec((NC, ob, D), lambda i: (0, i, 0)),
            pl.BlockSpec((ob, D), lambda i: (i, 0)),
            pl.BlockSpec((ob, NC), lambda i: (i, 0)),
            pl.BlockSpec((1, D), lambda i: (0, 0)),
        ],
        out_specs=pl.BlockSpec((ob, D), lambda i: (i, 0)),
        out_shape=jax.ShapeDtypeStruct((n_nodes, D), jnp.float32),
    )


@jax.jit
def kernel(x, edge_index, weight, bias):
    n = x.shape[0]
    e = edge_index.shape[1]
    row = edge_index[0].astype(jnp.int32)
    col = edge_index[1].astype(jnp.int32)
    epb = NW * CHUNK
    e_pad = ((e + epb - 1) // epb) * epb
    padv = jnp.full((e_pad - e + 8 * CHUNK,), n, jnp.int32)
    row2d = jnp.concatenate([row, padv]).reshape(-1, CHUNK)
    col2d = jnp.concatenate([col, padv]).reshape(-1, CHUNK)
    x_p = jnp.zeros((N_PAD, x.shape[1]), jnp.float32).at[:n, :].set(x)

    zerosd = jnp.zeros((CHUNK, D), jnp.float32)

    degbuf = _sc_degree(e_pad)(row2d.reshape(-1))
    db = jnp.transpose(degbuf.reshape(NC, N_PAD))
    g = _tc_g(x_p, weight, db)
    s = _sc_segsum(e_pad)(g, row2d.reshape(-1), col2d.reshape(-1), zerosd)
    out = _tc_out(n, 2000)(s, g, db, bias.reshape(1, D))
    return out


# speedup vs baseline: 1.1292x; 1.1292x over previous
import functools

import jax
import jax.numpy as jnp
from jax import lax
from jax.experimental import pallas as pl
from jax.experimental.pallas import tpu as pltpu
from jax.experimental.pallas import tpu_sc as plsc

NC = 2
NS = 16
L = 16
NW = NC * NS

D = 128
CHUNK = 128
MB = 1280
N_PAD = 10240
RPS = N_PAD // NS


def _mesh():
    return plsc.VectorSubcoreMesh(
        core_axis_name="c", subcore_axis_name="s", num_cores=NC, num_subcores=NS
    )


@functools.lru_cache(maxsize=None)
def _sc_degree(e_pad):
    ept = e_pad // NW
    nch = ept // CHUNK

    @functools.partial(
        pl.kernel,
        out_type=jax.ShapeDtypeStruct((NC * N_PAD,), jnp.float32),
        mesh=_mesh(),
        scratch_types=[
            pltpu.VMEM((CHUNK,), jnp.int32),
            pltpu.VMEM((CHUNK,), jnp.float32),
            pltpu.VMEM((RPS,), jnp.float32),
            pltpu.VMEM_SHARED((N_PAD,), jnp.float32),
        ],
    )
    def deg_kernel(row_hbm, out_hbm, idx_v, ones_v, zero_v, s1):
        c = lax.axis_index("c")
        s = lax.axis_index("s")
        wid = s * NC + c
        ones = jnp.ones((L,), jnp.float32)
        zeros = jnp.zeros((L,), jnp.float32)
        for j in range(CHUNK // L):
            ones_v[pl.ds(j * L, L)] = ones

        def zfill(i, _):
            zero_v[pl.ds(i * L, L)] = zeros
            return 0

        lax.fori_loop(0, RPS // L, zfill, 0)
        pltpu.sync_copy(zero_v, s1.at[pl.ds(s * RPS, RPS)])
        plsc.subcore_barrier()

        def chunk(j, _):
            off = pl.multiple_of(wid * ept + j * CHUNK, CHUNK)
            pltpu.sync_copy(row_hbm.at[pl.ds(off, CHUNK)], idx_v)
            pltpu.sync_copy(ones_v, s1.at[idx_v], add=True)
            return 0

        lax.fori_loop(0, nch, chunk, 0)
        plsc.subcore_barrier()
        pltpu.sync_copy(
            s1.at[pl.ds(s * RPS, RPS)],
            out_hbm.at[pl.ds(c * N_PAD + s * RPS, RPS)],
        )

    return deg_kernel


@functools.lru_cache(maxsize=None)
def _sc_segsum(e_pad):
    ept = e_pad // NW
    nch = ept // CHUNK

    @functools.partial(
        pl.kernel,
        out_type=jax.ShapeDtypeStruct((NC, N_PAD, D), jnp.float32),
        mesh=_mesh(),
        scratch_types=[
            pltpu.VMEM((CHUNK,), jnp.int32),
            pltpu.VMEM((CHUNK,), jnp.int32),
            pltpu.VMEM((CHUNK, D), jnp.float32),
            pltpu.VMEM_SHARED((N_PAD, D), jnp.float32),
            pltpu.SemaphoreType.DMA,
        ],
    )
    def seg_kernel(
        g_hbm, row_hbm, col_hbm, zeros_hbm, out_hbm, ridx_v, cidx_v, rows_v, sacc, sem
    ):
        c = lax.axis_index("c")
        s = lax.axis_index("s")
        wid = s * NC + c
        for j in range(RPS // CHUNK):
            pltpu.sync_copy(
                zeros_hbm, sacc.at[pl.ds(s * RPS + j * CHUNK, CHUNK)]
            )
        plsc.subcore_barrier()

        def chunk(j, _):
            off = pl.multiple_of(wid * ept + j * CHUNK, CHUNK)
            pltpu.sync_copy(row_hbm.at[pl.ds(off, CHUNK)], ridx_v)
            pltpu.sync_copy(col_hbm.at[pl.ds(off, CHUNK)], cidx_v)
            pltpu.async_copy(g_hbm.at[cidx_v], rows_v, sem).wait()
            pltpu.sync_copy(rows_v, sacc.at[ridx_v], add=True)
            return 0

        lax.fori_loop(0, nch, chunk, 0)
        plsc.subcore_barrier()
        pltpu.sync_copy(
            sacc.at[pl.ds(s * RPS, RPS)], out_hbm.at[c, pl.ds(s * RPS, RPS)]
        )

    return seg_kernel


def _tc_g_body(x_ref, w_ref, db_ref, g_ref):
    db = db_ref[...]
    deg = 1.0 + db[:, 0] + db[:, 1]
    d = lax.rsqrt(deg)
    h = jnp.dot(x_ref[...], w_ref[...], preferred_element_type=jnp.float32)
    g_ref[...] = h * d[:, None]


_tc_g = pl.pallas_call(
    _tc_g_body,
    grid=(N_PAD // MB,),
    in_specs=[
        pl.BlockSpec((MB, D), lambda i: (i, 0)),
        pl.BlockSpec((D, D), lambda i: (0, 0)),
        pl.BlockSpec((MB, NC), lambda i: (i, 0)),
    ],
    out_specs=pl.BlockSpec((MB, D), lambda i: (i, 0)),
    out_shape=jax.ShapeDtypeStruct((N_PAD, D), jnp.float32),
)


def _tc_out_body(s_ref, g_ref, db_ref, b_ref, o_ref):
    db = db_ref[...]
    deg = 1.0 + db[:, 0] + db[:, 1]
    d = lax.rsqrt(deg)
    sv = s_ref[...]
    tot = sv[0] + sv[1] + g_ref[...]
    o_ref[...] = tot * d[:, None] + b_ref[...]


def _tc_out(n_nodes, ob):
    return pl.pallas_call(
        _tc_out_body,
        grid=(n_nodes // ob,),
        in_specs=[
            pl.BlockSpec((NC, ob, D), lambda i: (0, i, 0)),
            pl.BlockSpec((ob, D), lambda i: (i, 0)),
            pl.BlockSpec((ob, NC), lambda i: (i, 0)),
            pl.BlockSpec((1, D), lambda i: (0, 0)),
        ],
        out_specs=pl.BlockSpec((ob, D), lambda i: (i, 0)),
        out_shape=jax.ShapeDtypeStruct((n_nodes, D), jnp.float32),
    )


@jax.jit
def kernel(x, edge_index, weight, bias):
    n = x.shape[0]
    e = edge_index.shape[1]
    row = edge_index[0].astype(jnp.int32)
    col = edge_index[1].astype(jnp.int32)
    epb = NW * CHUNK
    e_pad = ((e + epb - 1) // epb) * epb
    padv = jnp.full((e_pad - e,), n, jnp.int32)
    row_p = jnp.concatenate([row, padv])
    col_p = jnp.concatenate([col, padv])
    x_p = jnp.zeros((N_PAD, x.shape[1]), jnp.float32).at[:n, :].set(x)

    zerosd = jnp.zeros((CHUNK, D), jnp.float32)

    degbuf = _sc_degree(e_pad)(row_p)
    db = jnp.transpose(degbuf.reshape(NC, N_PAD))
    g = _tc_g(x_p, weight, db)
    s = _sc_segsum(e_pad)(g, row_p, col_p, zerosd)
    out = _tc_out(n, 2000)(s, g, db, bias.reshape(1, D))
    return out


# speedup vs baseline: 1.2071x; 1.0690x over previous
import functools

import jax
import jax.numpy as jnp
from jax import lax
from jax.experimental import pallas as pl
from jax.experimental.pallas import tpu as pltpu
from jax.experimental.pallas import tpu_sc as plsc

NC = 2
NS = 16
L = 16
NW = NC * NS

D = 128
CHUNK = 128
MB = 1280
N_PAD = 10240
RPS = N_PAD // NS


def _mesh():
    return plsc.VectorSubcoreMesh(
        core_axis_name="c", subcore_axis_name="s", num_cores=NC, num_subcores=NS
    )


@functools.lru_cache(maxsize=None)
def _sc_degree(e_pad):
    ept = e_pad // NW
    nch = ept // CHUNK

    @functools.partial(
        pl.kernel,
        out_type=jax.ShapeDtypeStruct((NC * N_PAD,), jnp.float32),
        mesh=_mesh(),
        scratch_types=[
            pltpu.VMEM((CHUNK,), jnp.int32),
            pltpu.VMEM((CHUNK,), jnp.float32),
            pltpu.VMEM((RPS,), jnp.float32),
            pltpu.VMEM_SHARED((N_PAD,), jnp.float32),
        ],
    )
    def deg_kernel(row_hbm, out_hbm, idx_v, ones_v, zero_v, s1):
        c = lax.axis_index("c")
        s = lax.axis_index("s")
        wid = s * NC + c
        ones = jnp.ones((L,), jnp.float32)
        zeros = jnp.zeros((L,), jnp.float32)
        for j in range(CHUNK // L):
            ones_v[pl.ds(j * L, L)] = ones

        def zfill(i, _):
            zero_v[pl.ds(i * L, L)] = zeros
            return 0

        lax.fori_loop(0, RPS // L, zfill, 0)
        pltpu.sync_copy(zero_v, s1.at[pl.ds(s * RPS, RPS)])
        plsc.subcore_barrier()

        def chunk(j, _):
            off = pl.multiple_of(wid * ept + j * CHUNK, CHUNK)
            pltpu.sync_copy(row_hbm.at[pl.ds(off, CHUNK)], idx_v)
            pltpu.sync_copy(ones_v, s1.at[idx_v], add=True)
            return 0

        lax.fori_loop(0, nch, chunk, 0)
        plsc.subcore_barrier()
        pltpu.sync_copy(
            s1.at[pl.ds(s * RPS, RPS)],
            out_hbm.at[pl.ds(c * N_PAD + s * RPS, RPS)],
        )

    return deg_kernel


@functools.lru_cache(maxsize=None)
def _sc_segsum(e_pad):
    ept = e_pad // NW
    nch = ept // CHUNK

    @functools.partial(
        pl.kernel,
        out_type=jax.ShapeDtypeStruct((NC, N_PAD, D), jnp.float32),
        mesh=_mesh(),
        scratch_types=[
            pltpu.VMEM((CHUNK,), jnp.int32),
            pltpu.VMEM((CHUNK,), jnp.int32),
            pltpu.VMEM((CHUNK, D), jnp.float32),
            pltpu.VMEM_SHARED((N_PAD, D), jnp.float32),
            pltpu.SemaphoreType.DMA,
            pltpu.SemaphoreType.DMA,
            pltpu.SemaphoreType.DMA,
        ],
    )
    def seg_kernel(
        g_hbm, row_hbm, col_hbm, zeros_hbm, out_hbm,
        ridx_v, cidx_v, rows_v, sacc, sem, si0, si1,
    ):
        c = lax.axis_index("c")
        s = lax.axis_index("s")
        wid = s * NC + c
        for j in range(RPS // CHUNK):
            pltpu.sync_copy(
                zeros_hbm, sacc.at[pl.ds(s * RPS + j * CHUNK, CHUNK)]
            )
        plsc.subcore_barrier()

        def chunk(j, _):
            off = pl.multiple_of(wid * ept + j * CHUNK, CHUNK)
            pltpu.async_copy(row_hbm.at[pl.ds(off, CHUNK)], ridx_v, si0)
            pltpu.async_copy(col_hbm.at[pl.ds(off, CHUNK)], cidx_v, si1)
            pltpu.make_async_copy(col_hbm.at[pl.ds(off, CHUNK)], cidx_v, si1).wait()
            pltpu.async_copy(g_hbm.at[cidx_v], rows_v, sem).wait()
            pltpu.make_async_copy(row_hbm.at[pl.ds(off, CHUNK)], ridx_v, si0).wait()
            pltpu.sync_copy(rows_v, sacc.at[ridx_v], add=True)
            return 0

        lax.fori_loop(0, nch, chunk, 0)
        plsc.subcore_barrier()
        pltpu.sync_copy(
            sacc.at[pl.ds(s * RPS, RPS)], out_hbm.at[c, pl.ds(s * RPS, RPS)]
        )

    return seg_kernel


def _tc_g_body(x_ref, w_ref, db_ref, g_ref):
    db = db_ref[...]
    deg = 1.0 + db[:, 0] + db[:, 1]
    d = lax.rsqrt(deg)
    h = jnp.dot(x_ref[...], w_ref[...], preferred_element_type=jnp.float32)
    g_ref[...] = h * d[:, None]


_tc_g = pl.pallas_call(
    _tc_g_body,
    grid=(N_PAD // MB,),
    in_specs=[
        pl.BlockSpec((MB, D), lambda i: (i, 0)),
        pl.BlockSpec((D, D), lambda i: (0, 0)),
        pl.BlockSpec((MB, NC), lambda i: (i, 0)),
    ],
    out_specs=pl.BlockSpec((MB, D), lambda i: (i, 0)),
    out_shape=jax.ShapeDtypeStruct((N_PAD, D), jnp.float32),
)


def _tc_out_body(s_ref, g_ref, db_ref, b_ref, o_ref):
    db = db_ref[...]
    deg = 1.0 + db[:, 0] + db[:, 1]
    d = lax.rsqrt(deg)
    sv = s_ref[...]
    tot = sv[0] + sv[1] + g_ref[...]
    o_ref[...] = tot * d[:, None] + b_ref[...]


def _tc_out(n_nodes, ob):
    return pl.pallas_call(
        _tc_out_body,
        grid=(n_nodes // ob,),
        in_specs=[
            pl.BlockSpec((NC, ob, D), lambda i: (0, i, 0)),
            pl.BlockSpec((ob, D), lambda i: (i, 0)),
            pl.BlockSpec((ob, NC), lambda i: (i, 0)),
            pl.BlockSpec((1, D), lambda i: (0, 0)),
        ],
        out_specs=pl.BlockSpec((ob, D), lambda i: (i, 0)),
        out_shape=jax.ShapeDtypeStruct((n_nodes, D), jnp.float32),
    )


@jax.jit
def kernel(x, edge_index, weight, bias):
    n = x.shape[0]
    e = edge_index.shape[1]
    row = edge_index[0].astype(jnp.int32)
    col = edge_index[1].astype(jnp.int32)
    epb = NW * CHUNK
    e_pad = ((e + epb - 1) // epb) * epb
    padv = jnp.full((e_pad - e,), n, jnp.int32)
    row_p = jnp.concatenate([row, padv])
    col_p = jnp.concatenate([col, padv])
    x_p = jnp.zeros((N_PAD, x.shape[1]), jnp.float32).at[:n, :].set(x)

    zerosd = jnp.zeros((CHUNK, D), jnp.float32)

    degbuf = _sc_degree(e_pad)(row_p)
    db = jnp.transpose(degbuf.reshape(NC, N_PAD))
    g = _tc_g(x_p, weight, db)
    s = _sc_segsum(e_pad)(g, row_p, col_p, zerosd)
    out = _tc_out(n, 2000)(s, g, db, bias.reshape(1, D))
    return out


# speedup vs baseline: 1.2282x; 1.0175x over previous
import functools

import jax
import jax.numpy as jnp
from jax import lax
from jax.experimental import pallas as pl
from jax.experimental.pallas import tpu as pltpu
from jax.experimental.pallas import tpu_sc as plsc

NC = 2
NS = 16
L = 16
NW = NC * NS

D = 128
CHUNK = 128
MB = 1280
N_PAD = 10240
RPS = N_PAD // NS


def _mesh():
    return plsc.VectorSubcoreMesh(
        core_axis_name="c", subcore_axis_name="s", num_cores=NC, num_subcores=NS
    )


@functools.lru_cache(maxsize=None)
def _sc_degree(e_pad):
    ept = e_pad // NW
    nch = ept // CHUNK

    @functools.partial(
        pl.kernel,
        out_type=jax.ShapeDtypeStruct((NC * N_PAD,), jnp.float32),
        mesh=_mesh(),
        scratch_types=[
            pltpu.VMEM((2, CHUNK), jnp.int32),
            pltpu.VMEM((CHUNK,), jnp.float32),
            pltpu.VMEM((RPS,), jnp.float32),
            pltpu.VMEM_SHARED((N_PAD,), jnp.float32),
            pltpu.SemaphoreType.DMA,
            pltpu.SemaphoreType.DMA,
        ],
    )
    def deg_kernel(row_hbm, out_hbm, idx2, ones_v, zero_v, s1, a0, a1):
        c = lax.axis_index("c")
        s = lax.axis_index("s")
        wid = s * NC + c
        ones = jnp.ones((L,), jnp.float32)
        zeros = jnp.zeros((L,), jnp.float32)
        for j in range(CHUNK // L):
            ones_v[pl.ds(j * L, L)] = ones

        def zfill(i, _):
            zero_v[pl.ds(i * L, L)] = zeros
            return 0

        lax.fori_loop(0, RPS // L, zfill, 0)
        pltpu.sync_copy(zero_v, s1.at[pl.ds(s * RPS, RPS)])
        plsc.subcore_barrier()

        def off(j):
            return pl.multiple_of(wid * ept + j * CHUNK, CHUNK)

        pltpu.async_copy(row_hbm.at[pl.ds(off(0), CHUNK)], idx2.at[0], a0)

        def pair(i, _):
            j = 2 * i
            pltpu.make_async_copy(
                row_hbm.at[pl.ds(off(j), CHUNK)], idx2.at[0], a0
            ).wait()
            pltpu.async_copy(row_hbm.at[pl.ds(off(j + 1), CHUNK)], idx2.at[1], a1)
            pltpu.sync_copy(ones_v, s1.at[idx2.at[0]], add=True)
            pltpu.make_async_copy(
                row_hbm.at[pl.ds(off(j + 1), CHUNK)], idx2.at[1], a1
            ).wait()
            pltpu.async_copy(row_hbm.at[pl.ds(off(j + 2), CHUNK)], idx2.at[0], a0)
            pltpu.sync_copy(ones_v, s1.at[idx2.at[1]], add=True)
            return 0

        lax.fori_loop(0, (nch - 1) // 2, pair, 0)
        pltpu.make_async_copy(
            row_hbm.at[pl.ds(off(nch - 1), CHUNK)], idx2.at[0], a0
        ).wait()
        pltpu.sync_copy(ones_v, s1.at[idx2.at[0]], add=True)
        plsc.subcore_barrier()
        pltpu.sync_copy(
            s1.at[pl.ds(s * RPS, RPS)],
            out_hbm.at[pl.ds(c * N_PAD + s * RPS, RPS)],
        )

    return deg_kernel


@functools.lru_cache(maxsize=None)
def _sc_segsum(e_pad):
    ept = e_pad // NW
    nch = ept // CHUNK

    @functools.partial(
        pl.kernel,
        out_type=jax.ShapeDtypeStruct((NC, N_PAD, D), jnp.float32),
        mesh=_mesh(),
        scratch_types=[
            pltpu.VMEM((CHUNK,), jnp.int32),
            pltpu.VMEM((CHUNK,), jnp.int32),
            pltpu.VMEM((CHUNK, D), jnp.float32),
            pltpu.VMEM_SHARED((N_PAD, D), jnp.float32),
            pltpu.SemaphoreType.DMA,
            pltpu.SemaphoreType.DMA,
            pltpu.SemaphoreType.DMA,
        ],
    )
    def seg_kernel(
        g_hbm, row_hbm, col_hbm, zeros_hbm, out_hbm,
        ridx_v, cidx_v, rows_v, sacc, sem, si0, si1,
    ):
        c = lax.axis_index("c")
        s = lax.axis_index("s")
        wid = s * NC + c
        for j in range(RPS // CHUNK):
            pltpu.sync_copy(
                zeros_hbm, sacc.at[pl.ds(s * RPS + j * CHUNK, CHUNK)]
            )
        plsc.subcore_barrier()

        def chunk(j, _):
            off = pl.multiple_of(wid * ept + j * CHUNK, CHUNK)
            pltpu.async_copy(row_hbm.at[pl.ds(off, CHUNK)], ridx_v, si0)
            pltpu.async_copy(col_hbm.at[pl.ds(off, CHUNK)], cidx_v, si1)
            pltpu.make_async_copy(col_hbm.at[pl.ds(off, CHUNK)], cidx_v, si1).wait()
            pltpu.async_copy(g_hbm.at[cidx_v], rows_v, sem).wait()
            pltpu.make_async_copy(row_hbm.at[pl.ds(off, CHUNK)], ridx_v, si0).wait()
            pltpu.sync_copy(rows_v, sacc.at[ridx_v], add=True)
            return 0

        lax.fori_loop(0, nch, chunk, 0)
        plsc.subcore_barrier()
        pltpu.sync_copy(
            sacc.at[pl.ds(s * RPS, RPS)], out_hbm.at[c, pl.ds(s * RPS, RPS)]
        )

    return seg_kernel


def _tc_g_body(x_ref, w_ref, db_ref, g_ref):
    db = db_ref[...]
    deg = 1.0 + db[:, 0] + db[:, 1]
    d = lax.rsqrt(deg)
    h = jnp.dot(x_ref[...], w_ref[...], preferred_element_type=jnp.float32)
    g_ref[...] = h * d[:, None]


_tc_g = pl.pallas_call(
    _tc_g_body,
    grid=(N_PAD // MB,),
    in_specs=[
        pl.BlockSpec((MB, D), lambda i: (i, 0)),
        pl.BlockSpec((D, D), lambda i: (0, 0)),
        pl.BlockSpec((MB, NC), lambda i: (i, 0)),
    ],
    out_specs=pl.BlockSpec((MB, D), lambda i: (i, 0)),
    out_shape=jax.ShapeDtypeStruct((N_PAD, D), jnp.float32),
)


def _tc_out_body(s_ref, g_ref, db_ref, b_ref, o_ref):
    db = db_ref[...]
    deg = 1.0 + db[:, 0] + db[:, 1]
    d = lax.rsqrt(deg)
    sv = s_ref[...]
    tot = sv[0] + sv[1] + g_ref[...]
    o_ref[...] = tot * d[:, None] + b_ref[...]


def _tc_out(n_nodes, ob):
    return pl.pallas_call(
        _tc_out_body,
        grid=(n_nodes // ob,),
        in_specs=[
            pl.BlockSpec((NC, ob, D), lambda i: (0, i, 0)),
            pl.BlockSpec((ob, D), lambda i: (i, 0)),
            pl.BlockSpec((ob, NC), lambda i: (i, 0)),
            pl.BlockSpec((1, D), lambda i: (0, 0)),
        ],
        out_specs=pl.BlockSpec((ob, D), lambda i: (i, 0)),
        out_shape=jax.ShapeDtypeStruct((n_nodes, D), jnp.float32),
    )


@jax.jit
def kernel(x, edge_index, weight, bias):
    n = x.shape[0]
    e = edge_index.shape[1]
    row = edge_index[0].astype(jnp.int32)
    col = edge_index[1].astype(jnp.int32)
    epb = NW * CHUNK
    e_pad = ((e + epb - 1) // epb) * epb
    padv = jnp.full((e_pad - e,), n, jnp.int32)
    row_p = jnp.concatenate([row, padv])
    col_p = jnp.concatenate([col, padv])
    x_p = jnp.zeros((N_PAD, x.shape[1]), jnp.float32).at[:n, :].set(x)

    zerosd = jnp.zeros((CHUNK, D), jnp.float32)

    degbuf = _sc_degree(e_pad)(row_p)
    db = jnp.transpose(degbuf.reshape(NC, N_PAD))
    g = _tc_g(x_p, weight, db)
    s = _sc_segsum(e_pad)(g, row_p, col_p, zerosd)
    out = _tc_out(n, 2000)(s, g, db, bias.reshape(1, D))
    return out
